# baseline (speedup 1.0000x reference)
import jax
import jax.numpy as jnp
from jax.experimental import pallas as pl


def _head_kernel(x_ref, wr_ref, wb_ref, wc_ref, rot_ref, betas_ref, cam_ref):
    xb = x_ref[...]
    rot_ref[...] = jnp.dot(xb, wr_ref[...], preferred_element_type=jnp.float32)
    betas_ref[...] = jnp.dot(xb, wb_ref[...], preferred_element_type=jnp.float32)
    cam_ref[...] = jnp.dot(xb, wc_ref[...], preferred_element_type=jnp.float32)


def kernel(x, pred_class, W_rot, W_betas, W_cam):
    stage, bs, nq, ch = x.shape
    n = stage * bs * nq
    xf = x.reshape(n, ch)

    blk = 2000
    grid = n // blk

    rot, betas, cam = pl.pallas_call(
        _head_kernel,
        grid=(grid,),
        in_specs=[
            pl.BlockSpec((blk, ch), lambda i: (i, 0)),
            pl.BlockSpec((ch, 216), lambda i: (0, 0)),
            pl.BlockSpec((ch, 10), lambda i: (0, 0)),
            pl.BlockSpec((ch, 3), lambda i: (0, 0)),
        ],
        out_specs=[
            pl.BlockSpec((blk, 216), lambda i: (i, 0)),
            pl.BlockSpec((blk, 10), lambda i: (i, 0)),
            pl.BlockSpec((blk, 3), lambda i: (i, 0)),
        ],
        out_shape=[
            jax.ShapeDtypeStruct((n, 216), jnp.float32),
            jax.ShapeDtypeStruct((n, 10), jnp.float32),
            jax.ShapeDtypeStruct((n, 3), jnp.float32),
        ],
    )(xf, W_rot, W_betas, W_cam)

    rotmat = rot.reshape(stage, bs, nq, 24, 3, 3)
    betas = betas.reshape(stage, bs, nq, 10)
    camera = cam.reshape(stage, bs, nq, 3)
    return (rotmat, betas, camera)


# speedup vs baseline: 1.4461x; 1.4461x over previous
import jax
import jax.numpy as jnp
from jax.experimental import pallas as pl


def _contract(x, w):
    return jax.lax.dot_general(
        x, w, (((2,), (0,)), ((), ())), preferred_element_type=jnp.float32
    )


def _head_kernel(x_ref, wr_ref, wb_ref, wc_ref, rot_ref, betas_ref, cam_ref):
    xb = x_ref[...]
    rot_ref[...] = _contract(xb, wr_ref[...])
    betas_ref[...] = _contract(xb, wb_ref[...])
    cam_ref[...] = _contract(xb, wc_ref[...])


def kernel(x, pred_class, W_rot, W_betas, W_cam):
    stage, bs, nq, ch = x.shape
    g = stage * bs
    x3 = x.reshape(g, nq, ch)

    blk = 8
    grid = g // blk

    rot, betas, cam = pl.pallas_call(
        _head_kernel,
        grid=(grid,),
        in_specs=[
            pl.BlockSpec((blk, nq, ch), lambda i: (i, 0, 0)),
            pl.BlockSpec((ch, 216), lambda i: (0, 0)),
            pl.BlockSpec((ch, 10), lambda i: (0, 0)),
            pl.BlockSpec((ch, 3), lambda i: (0, 0)),
        ],
        out_specs=[
            pl.BlockSpec((blk, nq, 216), lambda i: (i, 0, 0)),
            pl.BlockSpec((blk, nq, 10), lambda i: (i, 0, 0)),
            pl.BlockSpec((blk, nq, 3), lambda i: (i, 0, 0)),
        ],
        out_shape=[
            jax.ShapeDtypeStruct((g, nq, 216), jnp.float32),
            jax.ShapeDtypeStruct((g, nq, 10), jnp.float32),
            jax.ShapeDtypeStruct((g, nq, 3), jnp.float32),
        ],
    )(x3, W_rot, W_betas, W_cam)

    rotmat = rot.reshape(stage, bs, nq, 24, 3, 3)
    betas = betas.reshape(stage, bs, nq, 10)
    camera = cam.reshape(stage, bs, nq, 3)
    return (rotmat, betas, camera)


# speedup vs baseline: 1.4683x; 1.0153x over previous
import jax
import jax.numpy as jnp
from jax.experimental import pallas as pl


def _contract(x, w):
    return jax.lax.dot_general(
        x, w, (((3,), (0,)), ((), ())), preferred_element_type=jnp.float32
    )


def _head_kernel(x_ref, wr_ref, wb_ref, wc_ref, rot_ref, betas_ref, cam_ref):
    xb = x_ref[...]
    rot_ref[...] = _contract(xb, wr_ref[...])
    betas_ref[...] = _contract(xb, wb_ref[...])
    cam_ref[...] = _contract(xb, wc_ref[...])


def kernel(x, pred_class, W_rot, W_betas, W_cam):
    stage, bs, nq, ch = x.shape

    blk = 8
    grid = (stage, bs // blk)

    rot, betas, camera = pl.pallas_call(
        _head_kernel,
        grid=grid,
        in_specs=[
            pl.BlockSpec((1, blk, nq, ch), lambda i, j: (i, j, 0, 0)),
            pl.BlockSpec((ch, 216), lambda i, j: (0, 0)),
            pl.BlockSpec((ch, 10), lambda i, j: (0, 0)),
            pl.BlockSpec((ch, 3), lambda i, j: (0, 0)),
        ],
        out_specs=[
            pl.BlockSpec((1, blk, nq, 216), lambda i, j: (i, j, 0, 0)),
            pl.BlockSpec((1, blk, nq, 10), lambda i, j: (i, j, 0, 0)),
            pl.BlockSpec((1, blk, nq, 3), lambda i, j: (i, j, 0, 0)),
        ],
        out_shape=[
            jax.ShapeDtypeStruct((stage, bs, nq, 216), jnp.float32),
            jax.ShapeDtypeStruct((stage, bs, nq, 10), jnp.float32),
            jax.ShapeDtypeStruct((stage, bs, nq, 3), jnp.float32),
        ],
    )(x, W_rot, W_betas, W_cam)

    rotmat = rot.reshape(stage, bs, nq, 24, 3, 3)
    return (rotmat, betas, camera)


# speedup vs baseline: 3.3934x; 2.3112x over previous
import numpy as np
import jax
import jax.numpy as jnp
from jax.experimental import pallas as pl

_RC = np.arange(9)
_J = np.arange(24)
_ROT_PERM = (_J[None, :] * 9 + _RC[:, None]).reshape(-1)


def _head_kernel(x_ref, wr_ref, wb_ref, wc_ref, rot_ref, betas_ref, cam_ref):
    blk = x_ref.shape[1]
    for b in range(blk):
        xb = x_ref[0, b]
        rot_ref[0, b] = jax.lax.dot_general(
            wr_ref[...], xb, (((1,), (0,)), ((), ())),
            preferred_element_type=jnp.float32)
        betas_ref[0, :, b, :] = jax.lax.dot_general(
            wb_ref[...], xb, (((1,), (0,)), ((), ())),
            preferred_element_type=jnp.float32)
        cam_ref[0, :, b, :] = jax.lax.dot_general(
            wc_ref[...], xb, (((1,), (0,)), ((), ())),
            preferred_element_type=jnp.float32)


def kernel(x, pred_class, W_rot, W_betas, W_cam):
    stage, bs, nq, ch = x.shape

    xt = jnp.transpose(x, (0, 1, 3, 2))
    wr_t = W_rot.T[_ROT_PERM]
    wb_t = W_betas.T
    wc_t = W_cam.T

    blk = 8
    grid = (stage, bs // blk)

    rot, betas, cam = pl.pallas_call(
        _head_kernel,
        grid=grid,
        in_specs=[
            pl.BlockSpec((1, blk, ch, nq), lambda i, j: (i, j, 0, 0)),
            pl.BlockSpec((216, ch), lambda i, j: (0, 0)),
            pl.BlockSpec((10, ch), lambda i, j: (0, 0)),
            pl.BlockSpec((3, ch), lambda i, j: (0, 0)),
        ],
        out_specs=[
            pl.BlockSpec((1, blk, 216, nq), lambda i, j: (i, j, 0, 0)),
            pl.BlockSpec((1, 10, blk, nq), lambda i, j: (i, 0, j, 0)),
            pl.BlockSpec((1, 3, blk, nq), lambda i, j: (i, 0, j, 0)),
        ],
        out_shape=[
            jax.ShapeDtypeStruct((stage, bs, 216, nq), jnp.float32),
            jax.ShapeDtypeStruct((stage, 10, bs, nq), jnp.float32),
            jax.ShapeDtypeStruct((stage, 3, bs, nq), jnp.float32),
        ],
    )(xt, wr_t, wb_t, wc_t)

    rotmat = jnp.transpose(
        rot.reshape(stage, bs, 3, 3, 24, nq), (0, 1, 5, 4, 2, 3))
    betas = jnp.transpose(betas, (0, 2, 3, 1))
    camera = jnp.transpose(cam, (0, 2, 3, 1))
    return (rotmat, betas, camera)


# speedup vs baseline: 6.3154x; 1.8611x over previous
import numpy as np
import jax
import jax.numpy as jnp
from jax.experimental import pallas as pl

_RC = np.arange(9)
_J = np.arange(24)
_ROT_PERM = (_J[None, :] * 9 + _RC[:, None]).reshape(-1)


def _head_kernel(x_ref, wr_ref, wb_ref, wc_ref, rot_ref, betas_ref, cam_ref):
    blk = x_ref.shape[2]
    for b in range(blk):
        xb = x_ref[0, :, b, :]
        rot_ref[0, b] = jax.lax.dot_general(
            wr_ref[...], xb, (((1,), (1,)), ((), ())),
            preferred_element_type=jnp.float32)
        betas_ref[0, :, b, :] = jax.lax.dot_general(
            wb_ref[...], xb, (((1,), (1,)), ((), ())),
            preferred_element_type=jnp.float32)
        cam_ref[0, :, b, :] = jax.lax.dot_general(
            wc_ref[...], xb, (((1,), (1,)), ((), ())),
            preferred_element_type=jnp.float32)


def kernel(x, pred_class, W_rot, W_betas, W_cam):
    stage, bs, nq, ch = x.shape

    xv = jnp.transpose(x, (0, 2, 1, 3))
    wr_t = W_rot.T[_ROT_PERM]
    wb_t = W_betas.T
    wc_t = W_cam.T

    blk = 8
    grid = (stage, bs // blk)

    rot, betas, cam = pl.pallas_call(
        _head_kernel,
        grid=grid,
        in_specs=[
            pl.BlockSpec((1, nq, blk, ch), lambda i, j: (i, 0, j, 0)),
            pl.BlockSpec((216, ch), lambda i, j: (0, 0)),
            pl.BlockSpec((10, ch), lambda i, j: (0, 0)),
            pl.BlockSpec((3, ch), lambda i, j: (0, 0)),
        ],
        out_specs=[
            pl.BlockSpec((1, blk, 216, nq), lambda i, j: (i, j, 0, 0)),
            pl.BlockSpec((1, 10, blk, nq), lambda i, j: (i, 0, j, 0)),
            pl.BlockSpec((1, 3, blk, nq), lambda i, j: (i, 0, j, 0)),
        ],
        out_shape=[
            jax.ShapeDtypeStruct((stage, bs, 216, nq), jnp.float32),
            jax.ShapeDtypeStruct((stage, 10, bs, nq), jnp.float32),
            jax.ShapeDtypeStruct((stage, 3, bs, nq), jnp.float32),
        ],
    )(xv, wr_t, wb_t, wc_t)

    rotmat = jnp.transpose(
        rot.reshape(stage, bs, 3, 3, 24, nq), (0, 1, 5, 4, 2, 3))
    betas = jnp.transpose(betas, (0, 2, 3, 1))
    camera = jnp.transpose(cam, (0, 2, 3, 1))
    return (rotmat, betas, camera)
